# baseline (speedup 1.0000x reference)
import functools

import jax
import jax.numpy as jnp
from jax import lax
from jax.experimental import pallas as pl
from jax.experimental.pallas import tpu as pltpu
from jax.experimental.pallas import tpu_sc as plsc

NU = 10000
NI = 10000
NN = 10000
E = 160000
HID = 256
H = 8
D = 32
HD = H * D

NC = 2
NS = 16
LANES = 16

ROW = 144
NP = 10240
TSTRIDE = 624
FCH = 32
NFCH = 640 // FCH
EPT = E // NS
ECH = 80
NSLOT = 2
IBLK = ECH * NSLOT
ITERS = EPT // IBLK

_mesh = plsc.VectorSubcoreMesh(core_axis_name="c", subcore_axis_name="s")
_sc_params = pltpu.CompilerParams(use_tc_tiling_on_sc=False)


def _bcast_lane(v, lane):
    idx = jnp.full((LANES, 1), lane, dtype=jnp.int32)
    dn = lax.GatherDimensionNumbers(
        offset_dims=(), collapsed_slice_dims=(0,), start_index_map=(0,))
    return lax.gather(v, idx, dn, (1,),
                      mode=lax.GatherScatterMode.PROMISE_IN_BOUNDS)


GSTR = 312
GCHS = (128, 128, 72)


@functools.partial(
    pl.kernel, mesh=_mesh, compiler_params=_sc_params,
    out_type=[
        jax.ShapeDtypeStruct((2 * NN, 128), jnp.float32),
        jax.ShapeDtypeStruct((2 * NN, 128), jnp.float32),
    ],
    scratch_types=[
        pltpu.VMEM((128,), jnp.int32),
        pltpu.VMEM((128,), jnp.int32),
        pltpu.VMEM((72,), jnp.int32),
        pltpu.VMEM((72,), jnp.int32),
        pltpu.VMEM((128, HID), jnp.float32),
        pltpu.VMEM((128, HID), jnp.float32),
        pltpu.VMEM((72, HID), jnp.float32),
        pltpu.VMEM((72, HID), jnp.float32),
        pltpu.SemaphoreType.DMA,
        pltpu.SemaphoreType.DMA,
        pltpu.SemaphoreType.DMA,
        pltpu.SemaphoreType.DMA,
    ],
)
def _sc_gather(uemb, iemb, ugid, igid, outl, outr,
               idxf0, idxf1, idxs0, idxs1, rowf0, rowf1, rows0, rows1,
               g0, g1, w0, w1):
    c = lax.axis_index("c")
    s = lax.axis_index("s")
    w = s * NC + c
    wbase = w * GSTR
    idxf = (idxf0, idxf1)
    idxs = (idxs0, idxs1)
    rowf = (rowf0, rowf1)
    rowsm = (rows0, rows1)
    gsem = (g0, g1)
    wsem = (w0, w1)

    jobs = []
    for side in range(2):
        for k, ln in enumerate(GCHS):
            jobs.append((side, side * NN, k, ln))

    def _writeout(rb, obase, ln, sem):
        d1 = pltpu.async_copy(rb.at[pl.ds(0, ln), pl.ds(0, 128)],
                              outl.at[pl.ds(obase, ln)], sem)
        d2 = pltpu.async_copy(rb.at[pl.ds(0, ln), pl.ds(128, 128)],
                              outr.at[pl.ds(obase, ln)], sem)
        return (d1, d2)

    gd = [None, None]
    wd = [None, None]
    meta = [None, None]
    for j, (side, ooff, k, ln) in enumerate(jobs):
        b = j % 2
        emb = uemb if side == 0 else iemb
        gid = ugid if side == 0 else igid
        ib = idxf[b] if ln == 128 else idxs[b]
        rb = rowf[b] if ln == 128 else rowsm[b]
        if wd[b] is not None:
            wd[b][0].wait()
            wd[b][1].wait()
        pltpu.sync_copy(gid.at[pl.ds(wbase + k * 128, ln)], ib)
        gd[b] = pltpu.async_copy(emb.at[ib], rb, gsem[b])
        meta[b] = (rb, ooff + wbase + k * 128, ln)
        if j >= 1:
            pb = 1 - b
            gd[pb].wait()
            prb, obase, oln = meta[pb]
            wd[pb] = _writeout(prb, obase, oln, wsem[pb])
    lb = (len(jobs) - 1) % 2
    gd[lb].wait()
    prb, obase, oln = meta[lb]
    for dsc in _writeout(prb, obase, oln, gsem[lb]):
        dsc.wait()
    if wd[1 - lb] is not None:
        wd[1 - lb][0].wait()
        wd[1 - lb][1].wait()


TB = 1000
TNB = NN // TB


def _tc_transform_body(h0l_ref, h0r_ref, ws_ref, wd_ref, alf_ref, arf_ref,
                       f_ref, l_ref, e_ref, yscr):
    p = pl.program_id(1)

    @pl.when(p == 0)
    def _():
        h0 = jnp.concatenate([h0l_ref[...], h0r_ref[...]], axis=1)
        yscr[:, :HD] = jnp.dot(h0, ws_ref[...],
                               preferred_element_type=jnp.float32)
        yscr[:, HD:] = jnp.dot(h0, wd_ref[...],
                               preferred_element_type=jnp.float32)

    hs = yscr[:, :HD]
    hd = yscr[:, HD:]
    ri = lax.broadcasted_iota(jnp.int32, (HD, H), 0) // D
    ci = lax.broadcasted_iota(jnp.int32, (HD, H), 1)
    mask = (ri == ci).astype(jnp.float32)
    el = jnp.dot(hs * alf_ref[...], mask, preferred_element_type=jnp.float32)

    @pl.when(p == 0)
    def _():
        er = jnp.dot(hd * arf_ref[...], mask,
                     preferred_element_type=jnp.float32)
        e_ref[...] = jnp.concatenate([er, jnp.zeros((TB, 8), jnp.float32)],
                                     axis=1)
        f_ref[...] = hs[:, :128]
        l_ref[...] = jnp.concatenate(
            [el[:, :4], jnp.zeros((TB, 12), jnp.float32)], axis=1)

    @pl.when(p == 1)
    def _():
        f_ref[...] = hs[:, 128:]
        l_ref[...] = jnp.concatenate(
            [jnp.zeros((TB, 4), jnp.float32), el[:, 4:],
             jnp.zeros((TB, 8), jnp.float32)], axis=1)


def _tc_transform(h0l, h0r, w_src, w_dst, al_flat, ar_flat, side):
    return pl.pallas_call(
        _tc_transform_body,
        grid=(TNB, 2),
        in_specs=[
            pl.BlockSpec((TB, 128), lambda i, p: (side * TNB + i, 0)),
            pl.BlockSpec((TB, 128), lambda i, p: (side * TNB + i, 0)),
            pl.BlockSpec((HID, HD), lambda i, p: (0, 0)),
            pl.BlockSpec((HID, HD), lambda i, p: (0, 0)),
            pl.BlockSpec((1, HD), lambda i, p: (0, 0)),
            pl.BlockSpec((1, HD), lambda i, p: (0, 0)),
        ],
        out_specs=[
            pl.BlockSpec((TB, 128), lambda i, p: (p * TNB + i, 0)),
            pl.BlockSpec((TB, 16), lambda i, p: (p * TNB + i, 0)),
            pl.BlockSpec((TB, 16), lambda i, p: (i, 0)),
        ],
        out_shape=[
            jax.ShapeDtypeStruct((2 * NN, 128), jnp.float32),
            jax.ShapeDtypeStruct((2 * NN, 16), jnp.float32),
            jax.ShapeDtypeStruct((NN, 16), jnp.float32),
        ],
        scratch_shapes=[pltpu.VMEM((TB, 2 * HD), jnp.float32)],
    )(h0l, h0r, w_src, w_dst, al_flat, ar_flat)


@functools.partial(
    pl.kernel, mesh=_mesh, compiler_params=_sc_params,
    out_type=[
        jax.ShapeDtypeStruct((2 * NN, 128), jnp.float32),
        jax.ShapeDtypeStruct((2 * NN, 16), jnp.float32),
        jax.ShapeDtypeStruct((2 * NN, 128), jnp.float32),
        jax.ShapeDtypeStruct((2 * NN, 16), jnp.float32),
    ],
    scratch_types=(
        [pltpu.VMEM_SHARED((NP, 128), jnp.float32)]
        + [pltpu.VMEM_SHARED((NP, 16), jnp.float32)]
        + [pltpu.VMEM((IBLK,), jnp.int32)] * 2
        + [pltpu.VMEM((ECH,), jnp.int32)] * NSLOT
        + [pltpu.VMEM((ECH,), jnp.int32)] * NSLOT
        + [pltpu.VMEM((ECH, 128), jnp.float32)] * NSLOT
        + [pltpu.VMEM((ECH, 16), jnp.float32)] * NSLOT
        + [pltpu.VMEM((ECH, 16), jnp.float32)] * NSLOT
        + [pltpu.VMEM((ECH, 16), jnp.float32)] * NSLOT
        + [pltpu.VMEM((FCH, 128), jnp.float32)]
        + [pltpu.VMEM((FCH, 16), jnp.float32)]
        + [pltpu.SemaphoreType.DMA] * NSLOT
        + [pltpu.SemaphoreType.DMA] * NSLOT
        + [pltpu.SemaphoreType.DMA]
    ),
)
def _sc_edges(esrc, edst, afeat, bfeat, ael, bel, erit, errev,
              nout_it, dout_it, nout_rev, dout_rev,
              accf, accd, sbig, dbig, *rest):
    sidx = rest[0:NSLOT]
    didx = rest[NSLOT:2 * NSLOT]
    featb = rest[2 * NSLOT:3 * NSLOT]
    elb = rest[3 * NSLOT:4 * NSLOT]
    erb = rest[4 * NSLOT:5 * NSLOT]
    wbuf = rest[5 * NSLOT:6 * NSLOT]
    zf = rest[6 * NSLOT]
    zd = rest[6 * NSLOT + 1]
    gsem = rest[6 * NSLOT + 2:7 * NSLOT + 2]
    ssem = rest[7 * NSLOT + 2:8 * NSLOT + 2]
    isem = rest[8 * NSLOT + 2]

    c = lax.axis_index("c")
    s = lax.axis_index("s")
    coff = c * NN
    lane0 = c * 4

    def _zrow(r, carry):
        for j in range(8):
            zf[r, pl.ds(j * LANES, LANES)] = jnp.zeros((LANES,), jnp.float32)
        zd[r, :] = jnp.zeros((LANES,), jnp.float32)
        return carry
    lax.fori_loop(0, FCH, _zrow, 0)

    def _edges_of(featb_b, elb_b, erb_b, wbuf_b):
        def _edge(e0, cy):
            for u in range(2):
                e = e0 * 2 + u
                x = elb_b[e, :] + erb_b[e, :]
                x = jnp.maximum(x, x * jnp.float32(0.2))
                w = jnp.exp(x)
                wbuf_b[e, :] = w
                for h in range(4):
                    wb = _bcast_lane(w, lane0 + h)
                    s0 = pl.ds(h * 32, LANES)
                    s1 = pl.ds(h * 32 + LANES, LANES)
                    featb_b[e, s0] = featb_b[e, s0] * wb
                    featb_b[e, s1] = featb_b[e, s1] * wb
            return cy
        lax.fori_loop(0, ECH // 2, _edge, 0)

    def _scat_wait(b):
        pltpu.make_async_copy(featb[b], accf.at[didx[b]], ssem[b]).wait()
        pltpu.make_async_copy(wbuf[b], accd.at[didx[b]], ssem[b]).wait()

    def _scat_issue(b):
        pltpu.async_copy(featb[b], accf.at[didx[b]], ssem[b], add=True)
        pltpu.async_copy(wbuf[b], accd.at[didx[b]], ssem[b], add=True)

    for d in range(2):
        src_hbm = esrc if d == 0 else edst
        dst_hbm = edst if d == 0 else esrc
        ftab = afeat if d == 0 else bfeat
        ltab = ael if d == 0 else bel
        etab = erit if d == 0 else errev
        onum = nout_it if d == 0 else nout_rev
        oden = dout_it if d == 0 else dout_rev

        r0 = s * TSTRIDE
        for k in range(NFCH):
            pltpu.sync_copy(zf, accf.at[pl.ds(r0 + k * FCH, FCH)])
            pltpu.sync_copy(zd, accd.at[pl.ds(r0 + k * FCH, FCH)])
        plsc.subcore_barrier()

        pltpu.async_copy(src_hbm.at[pl.ds(s * EPT, IBLK)], sbig, isem)
        pltpu.async_copy(dst_hbm.at[pl.ds(s * EPT, IBLK)], dbig, isem)

        def _iter(j, carry):
            pltpu.make_async_copy(
                src_hbm.at[pl.ds(s * EPT, IBLK)], sbig, isem).wait()
            pltpu.make_async_copy(
                dst_hbm.at[pl.ds(s * EPT, IBLK)], dbig, isem).wait()
            gd = []
            for b in range(NSLOT):
                @pl.when(j > 0)
                def _():
                    _scat_wait(b)
                for v in range(ECH // LANES):
                    sl16 = pl.ds(b * ECH + v * LANES, LANES)
                    dsl = pl.ds(v * LANES, LANES)
                    sidx[b][dsl] = sbig[sl16] + coff
                    didx[b][dsl] = dbig[sl16]
                g1 = pltpu.async_copy(ftab.at[sidx[b]], featb[b], gsem[b])
                g2 = pltpu.async_copy(ltab.at[sidx[b]], elb[b], gsem[b])
                g3 = pltpu.async_copy(etab.at[didx[b]], erb[b], gsem[b])
                gd.append((g1, g2, g3))

            @pl.when(j < ITERS - 1)
            def _():
                nbase = s * EPT + (j + 1) * IBLK
                pltpu.async_copy(src_hbm.at[pl.ds(nbase, IBLK)], sbig, isem)
                pltpu.async_copy(dst_hbm.at[pl.ds(nbase, IBLK)], dbig, isem)

            for b in range(NSLOT):
                for g in gd[b]:
                    g.wait()
                _edges_of(featb[b], elb[b], erb[b], wbuf[b])
                _scat_issue(b)
            return carry
        lax.fori_loop(0, ITERS, _iter, 0)

        tbase = s * EPT + ITERS * IBLK
        _scat_wait(0)
        pltpu.sync_copy(src_hbm.at[pl.ds(tbase, ECH)], sidx[0])
        pltpu.sync_copy(dst_hbm.at[pl.ds(tbase, ECH)], didx[0])
        for v in range(ECH // LANES):
            sl = pl.ds(v * LANES, LANES)
            sidx[0][sl] = sidx[0][sl] + coff
        pltpu.async_copy(ftab.at[sidx[0]], featb[0], gsem[0]).wait()
        pltpu.async_copy(ltab.at[sidx[0]], elb[0], gsem[0]).wait()
        pltpu.async_copy(etab.at[didx[0]], erb[0], gsem[0]).wait()
        _edges_of(featb[0], elb[0], erb[0], wbuf[0])
        _scat_issue(0)

        for b in range(NSLOT):
            _scat_wait(b)
        plsc.subcore_barrier()

        for k in range(NFCH):
            rr = r0 + k * FCH
            pltpu.sync_copy(accf.at[pl.ds(rr, FCH)], zf)
            pltpu.sync_copy(zf, onum.at[pl.ds(coff + rr, FCH)])
            pltpu.sync_copy(accd.at[pl.ds(rr, FCH)], zd)
            pltpu.sync_copy(zd, oden.at[pl.ds(coff + rr, FCH)])

        if d == 0:
            lax.fori_loop(0, FCH, _zrow, 0)


def _tc_final_body(nit0_ref, nit1_ref, dit0_ref, dit1_ref,
                   nrev0_ref, nrev1_ref, drev0_ref, drev1_ref,
                   bit_ref, brev_ref, out_ref, acc_ref):
    p = pl.program_id(0)
    i = pl.program_id(1)
    ri = lax.broadcasted_iota(jnp.int32, (H, HD), 0)
    ci = lax.broadcasted_iota(jnp.int32, (H, HD), 1) // D
    smat = (ri == ci).astype(jnp.float32)

    @pl.when(p == 0)
    def _():
        den8 = jnp.concatenate(
            [dit0_ref[:, 0:4], dit1_ref[:, 4:8]], axis=1)
        rep = jnp.dot(den8, smat, preferred_element_type=jnp.float32)
        hit = (jnp.concatenate([nit0_ref[...], nit1_ref[...]], axis=1)
               / jnp.maximum(rep, 1e-9))
        colsum = jnp.sum(hit, axis=0, keepdims=True)

        @pl.when(i == 0)
        def _():
            acc_ref[0:1, :] = colsum

        @pl.when(i != 0)
        def _():
            acc_ref[0:1, :] = acc_ref[0:1, :] + colsum

    @pl.when(p == 1)
    def _():
        den8 = jnp.concatenate(
            [drev0_ref[:, 0:4], drev1_ref[:, 4:8]], axis=1)
        rep = jnp.dot(den8, smat, preferred_element_type=jnp.float32)
        left = (jnp.concatenate([nrev0_ref[...], nrev1_ref[...]], axis=1)
                / jnp.maximum(rep, 1e-9)) + brev_ref[...]
        mean = acc_ref[0:1, :] * jnp.float32(1.0 / NN) + bit_ref[...]
        out_ref[...] = jnp.concatenate(
            [left, jnp.broadcast_to(mean, (TB, HD))], axis=1)


def _tc_final(nit, dit, nrev, drev, b_it, b_rev):
    nblk0 = pl.BlockSpec((TB, 128), lambda p, i: (i, 0))
    nblk1 = pl.BlockSpec((TB, 128), lambda p, i: (i + TNB, 0))
    dblk0 = pl.BlockSpec((TB, 16), lambda p, i: (i, 0))
    dblk1 = pl.BlockSpec((TB, 16), lambda p, i: (i + TNB, 0))
    fixed = pl.BlockSpec((1, HD), lambda p, i: (0, 0))
    return pl.pallas_call(
        _tc_final_body,
        grid=(2, TNB),
        in_specs=[nblk0, nblk1, dblk0, dblk1,
                  nblk0, nblk1, dblk0, dblk1, fixed, fixed],
        out_specs=pl.BlockSpec((TB, 2 * HD), lambda p, i: (i, 0)),
        out_shape=jax.ShapeDtypeStruct((NN, 2 * HD), jnp.float32),
        scratch_shapes=[pltpu.VMEM((8, HD), jnp.float32)],
    )(nit, nit, dit, dit, nrev, nrev, drev, drev, b_it, b_rev)


def kernel(u_gid, i_gid, edge_src, edge_dst, user_emb, item_emb,
           W_it, al_it, ar_it, b_it, W_rev, al_rev, ar_rev, b_rev):
    u_gid = u_gid.astype(jnp.int32)
    i_gid = i_gid.astype(jnp.int32)
    edge_src = edge_src.astype(jnp.int32)
    edge_dst = edge_dst.astype(jnp.int32)

    h0l, h0r = _sc_gather(user_emb, item_emb, u_gid, i_gid)

    afeat, ael, er_rev = _tc_transform(
        h0l, h0r, W_it, W_rev,
        al_it.reshape(1, HD), ar_rev.reshape(1, HD), side=0)
    bfeat, bel, er_it = _tc_transform(
        h0l, h0r, W_rev, W_it,
        al_rev.reshape(1, HD), ar_it.reshape(1, HD), side=1)

    nit, dit, nrev, drev = _sc_edges(
        edge_src, edge_dst, afeat, bfeat, ael, bel, er_it, er_rev)

    return _tc_final(nit, dit, nrev, drev,
                     b_it.reshape(1, HD), b_rev.reshape(1, HD))
